# baseline (speedup 1.0000x reference)
import jax
import jax.numpy as jnp
from jax.experimental import pallas as pl
from jax.experimental.pallas import tpu as pltpu

_ATOM_VOCABS = (100, 10, 10, 10, 10)
_NUM_IDX = 5
_ATOM_BASES = tuple(int(sum(_ATOM_VOCABS[:i])) for i in range(_NUM_IDX))
_ATOM_TOTAL = int(sum(_ATOM_VOCABS))
_OUT_FEATURES = 256
_SPLIT = 96
_K_PAD = 144


def _round_up(v, m):
    return (v + m - 1) // m * m


def _node_embed_kernel(x_ref, w_ref, o_ref):
    xt = x_ref[...].T
    tn = xt.shape[1]
    xi = xt[:_NUM_IDX, :].astype(jnp.int32)
    feat = xt[_NUM_IDX:_NUM_IDX + 1, :]

    iota_a = jax.lax.broadcasted_iota(jnp.int32, (_SPLIT, tn), 0)
    lhs_a = jnp.where(iota_a == xi[0:1, :], 1.0, 0.0)

    iota_b = jax.lax.broadcasted_iota(jnp.int32, (_K_PAD - _SPLIT, tn), 0) + _SPLIT
    mask = iota_b == xi[0:1, :]
    for i in range(1, _NUM_IDX):
        mask = mask | (iota_b == (xi[i:i + 1, :] + _ATOM_BASES[i]))
    mask = mask | (iota_b == (_ATOM_TOTAL + 1))
    lhs_b = jnp.where(mask, 1.0, 0.0)
    lhs_b = jnp.where(iota_b == _ATOM_TOTAL, feat, lhs_b)

    lhs_t = jnp.concatenate([lhs_a, lhs_b], axis=0)
    o_ref[...] = jax.lax.dot_general(
        lhs_t, w_ref[...],
        dimension_numbers=(((0,), (0,)), ((), ())),
        preferred_element_type=jnp.float32)


def _node_embed_forward(x, w_node, *, tile_n=8192):
    n, f = x.shape
    out_pad = int(w_node.shape[1])
    w_pad = jnp.pad(w_node, ((0, _K_PAD - w_node.shape[0]), (0, 0)))

    tile = min(tile_n, _round_up(n, 8))
    n_pad = _round_up(n, tile)
    if n_pad != n:
        x = jnp.pad(x, ((0, n_pad - n), (0, 0)))

    out = pl.pallas_call(
        _node_embed_kernel,
        out_shape=jax.ShapeDtypeStruct((n_pad, out_pad), jnp.float32),
        grid=(n_pad // tile,),
        in_specs=[
            pl.BlockSpec((tile, f), lambda i: (i, 0)),
            pl.BlockSpec((_K_PAD, out_pad), lambda i: (0, 0)),
        ],
        out_specs=pl.BlockSpec((tile, out_pad), lambda i: (i, 0)),
        compiler_params=pltpu.CompilerParams(
            dimension_semantics=("parallel",)),
    )(x, w_pad)
    return out[:n, :_OUT_FEATURES]


def kernel(x, edge_attr, w_node):
    del edge_attr
    return _node_embed_forward(x, w_node)


# speedup vs baseline: 2.2254x; 2.2254x over previous
import jax
import jax.numpy as jnp
from jax.experimental import pallas as pl
from jax.experimental.pallas import tpu as pltpu

_ATOM_VOCABS = (100, 10, 10, 10, 10)
_NUM_IDX = 5
_ATOM_BASES = tuple(int(sum(_ATOM_VOCABS[:i])) for i in range(_NUM_IDX))
_ATOM_TOTAL = int(sum(_ATOM_VOCABS))
_OUT_FEATURES = 256
_SPLIT = 96
_K_PAD = 144


def _round_up(v, m):
    return (v + m - 1) // m * m


def _node_embed_kernel(xt_ref, w_ref, o_ref):
    xt = xt_ref[...]
    tn = xt.shape[1]
    xi = xt[:_NUM_IDX, :].astype(jnp.int32)
    feat = xt[_NUM_IDX:_NUM_IDX + 1, :].astype(jnp.float32)

    iota_a = jax.lax.broadcasted_iota(jnp.int32, (_SPLIT, tn), 0)
    lhs_a = jnp.where(iota_a == xi[0:1, :], 1.0, 0.0)

    iota_b = jax.lax.broadcasted_iota(jnp.int32, (_K_PAD - _SPLIT, tn), 0) + _SPLIT
    mask = iota_b == xi[0:1, :]
    for i in range(1, _NUM_IDX):
        mask = mask | (iota_b == (xi[i:i + 1, :] + _ATOM_BASES[i]))
    mask = mask | (iota_b == (_ATOM_TOTAL + 1))
    lhs_b = jnp.where(mask, 1.0, 0.0)
    lhs_b = jnp.where(iota_b == _ATOM_TOTAL, feat, lhs_b)

    lhs_t = jnp.concatenate([lhs_a, lhs_b], axis=0)
    o_ref[...] = jax.lax.dot_general(
        lhs_t, w_ref[...],
        dimension_numbers=(((0,), (0,)), ((), ())),
        preferred_element_type=jnp.float32)


def _node_embed_forward(x, w_node, *, tile_n=16384):
    n, f = x.shape
    out_pad = int(w_node.shape[1])
    w_pad = jnp.pad(w_node, ((0, _K_PAD - w_node.shape[0]), (0, 0)))
    xt = jnp.pad(x.T.astype(jnp.bfloat16), ((0, 8 - f), (0, 0)))

    tile = min(tile_n, _round_up(n, 8))
    n_pad = _round_up(n, tile)
    if n_pad != n:
        xt = jnp.pad(xt, ((0, 0), (0, n_pad - n)))

    out = pl.pallas_call(
        _node_embed_kernel,
        out_shape=jax.ShapeDtypeStruct((n_pad, out_pad), jnp.float32),
        grid=(n_pad // tile,),
        in_specs=[
            pl.BlockSpec((8, tile), lambda i: (0, i)),
            pl.BlockSpec((_K_PAD, out_pad), lambda i: (0, 0)),
        ],
        out_specs=pl.BlockSpec((tile, out_pad), lambda i: (i, 0)),
        compiler_params=pltpu.CompilerParams(
            dimension_semantics=("parallel",)),
    )(xt, w_pad)
    return out[:n, :_OUT_FEATURES]


def kernel(x, edge_attr, w_node):
    del edge_attr
    return _node_embed_forward(x, w_node)
